# baseline (speedup 1.0000x reference)
import functools

import jax
import jax.numpy as jnp
from jax import lax
from jax.experimental import pallas as pl
from jax.experimental.pallas import tpu as pltpu
from jax.experimental.pallas import tpu_sc as plsc

NUM_EMB = 1000000
DIM = 64
SEQ = 16384
NT = 32
GRP_SMALL = (NUM_EMB // 8) // NT
CHUNK = 256
NCH = 124
L = 16
ECAP = 1040
HCAP = 48
DUMP = SEQ
BIGROW = 2**30
IDS_SL = 2048


def _body(table_hbm, idx_hbm, out_hbm,
          ids_v, erow_v, epos_v, cka_v, ckb_v, hloc_v, hpos_v, hdat_v,
          sga, sgb, ssc):
    wid = lax.axis_index("s") * 2 + lax.axis_index("c")
    rbase = 8 * (wid * GRP_SMALL + jnp.minimum(wid, 8))
    shard_sz = 8 * GRP_SMALL + jnp.where(wid < 8, 8, 0)
    def collect_slice(s, cnt):
        pltpu.sync_copy(idx_hbm.at[pl.ds(s * IDS_SL, IDS_SL)], ids_v)

        def collect(k, cnt):
            v = ids_v[pl.ds(k * L, L)]
            pos = lax.iota(jnp.int32, L) + (s * IDS_SL + k * L)
            m = jnp.logical_and(v >= rbase, v < rbase + shard_sz)
            plsc.store_compressed(erow_v.at[pl.ds(cnt, L)], v - rbase, mask=m)
            plsc.store_compressed(epos_v.at[pl.ds(cnt, L)], pos, mask=m)
            return cnt + plsc.all_reduce_population_count(m)[0]

        return lax.fori_loop(0, IDS_SL // L, collect, cnt)

    cnt = lax.fori_loop(0, SEQ // IDS_SL, collect_slice, jnp.int32(0))
    erow_v[pl.ds(cnt, L)] = jnp.full((L,), BIGROW, jnp.int32)
    ngrp = (cnt + L - 1) // L

    for h in range(HCAP // L):
        hloc_v[pl.ds(h * L, L)] = jnp.zeros((L,), jnp.int32)

    def cstart(c):
        return pl.multiple_of(
            rbase + jnp.minimum(c * CHUNK, shard_sz - CHUNK), 8)

    def fire(c, buf, sem):
        pltpu.async_copy(table_hbm.at[pl.ds(cstart(c), CHUNK), :], buf, sem)

    def process(c, buf):
        cs = cstart(c)
        for h in range(HCAP // L):
            hpos_v[pl.ds(h * L, L)] = jnp.full((L,), DUMP, jnp.int32)

        def match(g, hcnt):
            r16 = erow_v[pl.ds(g * L, L)]
            p16 = epos_v[pl.ds(g * L, L)]
            loc = r16 - (cs - rbase)
            m = jnp.logical_and(loc >= 0, loc < CHUNK)
            plsc.store_compressed(hloc_v.at[pl.ds(hcnt, L)], loc, mask=m)
            plsc.store_compressed(hpos_v.at[pl.ds(hcnt, L)], p16, mask=m)
            return hcnt + plsc.all_reduce_population_count(m)[0]

        hcnt = lax.fori_loop(0, ngrp, match, jnp.int32(0))

        def hitcopy(g, _):
            l16 = hloc_v[pl.ds(g * L, L)]
            h16 = lax.iota(jnp.int32, L) + g * L
            for j in range(DIM):
                js = jnp.full((L,), j, jnp.int32)
                vals = plsc.load_gather(buf, [l16, js])
                plsc.store_scatter(hdat_v, [h16, js], vals)
            return ()

        lax.fori_loop(0, (hcnt + L - 1) // L, hitcopy, ())
        pltpu.async_copy(hdat_v, out_hbm.at[hpos_v], ssc).wait()

    fire(jnp.int32(0), cka_v, sga)

    def pair(t, _):
        ca = 2 * t
        fire(ca + 1, ckb_v, sgb)
        pltpu.make_async_copy(
            table_hbm.at[pl.ds(0, CHUNK), :], cka_v, sga).wait()
        process(ca, cka_v)
        fire(ca + 2, cka_v, sga)
        pltpu.make_async_copy(
            table_hbm.at[pl.ds(0, CHUNK), :], ckb_v, sgb).wait()
        process(ca + 1, ckb_v)
        return ()

    lax.fori_loop(0, NCH // 2, pair, ())
    pltpu.make_async_copy(table_hbm.at[pl.ds(0, CHUNK), :], cka_v, sga).wait()


def kernel(token_ids, embedding_matrix):
    mesh = plsc.VectorSubcoreMesh(core_axis_name="c", subcore_axis_name="s")
    k = pl.kernel(
        _body,
        mesh=mesh,
        out_type=jax.ShapeDtypeStruct((SEQ + 1, 2 * DIM), jnp.float32),
        scratch_types=[
            pltpu.VMEM((IDS_SL,), jnp.int32),
            pltpu.VMEM((ECAP,), jnp.int32),
            pltpu.VMEM((ECAP,), jnp.int32),
            pltpu.VMEM((CHUNK, DIM), jnp.float32),
            pltpu.VMEM((CHUNK, DIM), jnp.float32),
            pltpu.VMEM((HCAP,), jnp.int32),
            pltpu.VMEM((HCAP,), jnp.int32),
            pltpu.VMEM((HCAP, 2 * DIM), jnp.float32),
            pltpu.SemaphoreType.DMA,
            pltpu.SemaphoreType.DMA,
            pltpu.SemaphoreType.DMA,
        ],
        compiler_params=pltpu.CompilerParams(needs_layout_passes=False),
    )
    out128 = k(embedding_matrix, token_ids.astype(jnp.int32))
    return out128[:SEQ, :DIM]


# speedup vs baseline: 7.1550x; 7.1550x over previous
import functools

import jax
import jax.numpy as jnp
from jax import lax
from jax.experimental import pallas as pl
from jax.experimental.pallas import tpu as pltpu
from jax.experimental.pallas import tpu_sc as plsc

NUM_EMB = 1000000
HALF = NUM_EMB // 2
DIM = 64
SEQ = 16384
NT = 32
B_PER_W = SEQ // NT
HB = 256
L = 16


def _body(taba, tabb, idx_hbm, out_hbm,
          ids_v, ia_v, ib_v, pa_v, pb_v, ra_v, rb_v, sg, ss):
    wid = lax.axis_index("s") * 2 + lax.axis_index("c")
    base = wid * B_PER_W

    def half_batch(h, _):
        hb = base + h * HB
        pltpu.sync_copy(idx_hbm.at[pl.ds(hb, HB)], ids_v)

        def prep(k, _):
            v = ids_v[pl.ds(k * L, L)]
            pos = lax.iota(jnp.int32, L) + (hb + k * L)
            slot = lax.iota(jnp.int32, L) + (h * HB + k * L)
            in_a = v < HALF
            ia_v[pl.ds(k * L, L)] = jnp.minimum(v, HALF - 1)
            ib_v[pl.ds(k * L, L)] = jnp.minimum(
                jnp.maximum(v - HALF, 0), HALF - 1)
            dump = SEQ + (wid + slot * NT) % B_PER_W
            pa_v[pl.ds(k * L, L)] = jnp.where(in_a, pos, dump)
            pb_v[pl.ds(k * L, L)] = jnp.where(in_a, dump, pos)
            return ()

        lax.fori_loop(0, HB // L, prep, ())
        pltpu.async_copy(taba.at[ia_v], ra_v, sg)
        pltpu.async_copy(tabb.at[ib_v], rb_v, sg)
        pltpu.make_async_copy(taba.at[pl.ds(0, HB), :], ra_v, sg).wait()
        pltpu.make_async_copy(tabb.at[pl.ds(0, HB), :], rb_v, sg).wait()
        pltpu.async_copy(ra_v, out_hbm.at[pa_v], ss)
        pltpu.async_copy(rb_v, out_hbm.at[pb_v], ss)
        pltpu.make_async_copy(ra_v, out_hbm.at[pl.ds(0, HB), :], ss).wait()
        pltpu.make_async_copy(rb_v, out_hbm.at[pl.ds(0, HB), :], ss).wait()
        return ()

    lax.fori_loop(0, B_PER_W // HB, half_batch, ())


def kernel(token_ids, embedding_matrix):
    taba = embedding_matrix[:HALF]
    tabb = embedding_matrix[HALF:]
    mesh = plsc.VectorSubcoreMesh(core_axis_name="c", subcore_axis_name="s")
    k = pl.kernel(
        _body,
        mesh=mesh,
        out_type=jax.ShapeDtypeStruct((SEQ + B_PER_W, DIM), jnp.float32),
        scratch_types=[
            pltpu.VMEM((HB,), jnp.int32),
            pltpu.VMEM((HB,), jnp.int32),
            pltpu.VMEM((HB,), jnp.int32),
            pltpu.VMEM((HB,), jnp.int32),
            pltpu.VMEM((HB,), jnp.int32),
            pltpu.VMEM((HB, DIM), jnp.float32),
            pltpu.VMEM((HB, DIM), jnp.float32),
            pltpu.SemaphoreType.DMA,
            pltpu.SemaphoreType.DMA,
        ],
        compiler_params=pltpu.CompilerParams(use_tc_tiling_on_sc=False),
    )
    out = k(taba, tabb, token_ids.astype(jnp.int32))
    return out[:SEQ]


# speedup vs baseline: 21.2596x; 2.9713x over previous
import functools

import jax
import jax.numpy as jnp
from jax import lax
from jax.experimental import pallas as pl
from jax.experimental.pallas import tpu as pltpu
from jax.experimental.pallas import tpu_sc as plsc

NUM_EMB = 1000000
DIM = 64
SEQ = 16384
NUM_WORKERS = 32
B_PER_W = SEQ // NUM_WORKERS
FLIGHT = 32
NG = B_PER_W // FLIGHT
DEPTH = 4


def _body(table_hbm, idx_hbm, out_hbm, idx_v, rows_v, sem0, sem1):
    wid = lax.axis_index("s") * 2 + lax.axis_index("c")
    base = wid * B_PER_W
    pltpu.sync_copy(idx_hbm.at[pl.ds(base, B_PER_W)], idx_v)
    sems = (sem0, sem1)

    def fire_s(g, sem):
        gb = g * FLIGHT
        for v16 in range(FLIGHT // 16):
            vec = idx_v[pl.ds(gb + v16 * 16, 16)]
            for i in range(16):
                row = vec[i]
                pltpu.async_copy(
                    table_hbm.at[pl.ds(row, 1), :],
                    rows_v.at[pl.ds(gb + v16 * 16 + i, 1), :],
                    sem,
                )

    def drain_s(g, sem):
        pltpu.make_async_copy(
            table_hbm.at[pl.ds(0, FLIGHT), :],
            rows_v.at[pl.ds(g * FLIGHT, FLIGHT), :],
            sem,
        ).wait()

    for g in range(DEPTH):
        fire_s(g, sems[g % 2])

    def pair(p, _):
        g = 2 * p
        drain_s(g, sems[0])
        fire_s(g + DEPTH, sems[0])
        drain_s(g + 1, sems[1])
        fire_s(g + DEPTH + 1, sems[1])
        return ()

    lax.fori_loop(0, (NG - DEPTH) // 2, pair, ())
    for g in range(NG - DEPTH, NG):
        drain_s(g, sems[g % 2])
    pltpu.sync_copy(rows_v, out_hbm.at[pl.ds(base, B_PER_W)])


def kernel(token_ids, embedding_matrix):
    mesh = plsc.VectorSubcoreMesh(core_axis_name="c", subcore_axis_name="s")
    k = pl.kernel(
        _body,
        mesh=mesh,
        out_type=jax.ShapeDtypeStruct((SEQ, DIM), jnp.float32),
        scratch_types=[
            pltpu.VMEM((B_PER_W,), jnp.int32),
            pltpu.VMEM((B_PER_W, DIM), jnp.float32),
            pltpu.SemaphoreType.DMA,
            pltpu.SemaphoreType.DMA,
        ],
    )
    return k(embedding_matrix, token_ids.astype(jnp.int32))
